# Initial kernel scaffold; baseline (speedup 1.0000x reference)
import jax
import jax.numpy as jnp
from jax.experimental import pallas as pl


def kernel(x, edge_index, batch, embed, Wl1, bl1, Wr1, Wl2, bl2, Wr2, Wlin, blin):
    raise NotImplementedError("write your pallas kernel here")


# speedup vs baseline: 4.9332x; 4.9332x over previous
import functools

import jax
import jax.numpy as jnp
from jax import lax
from jax.experimental import pallas as pl
from jax.experimental.pallas import tpu as pltpu
from jax.experimental.pallas import tpu_sc as plsc

_NCORE = 2
_NSUB = 16
_BLK = 512


def _sc_mesh():
    return plsc.VectorSubcoreMesh(core_axis_name="c", subcore_axis_name="s")


_SC_PARAMS = pltpu.CompilerParams(use_tc_tiling_on_sc=False)


def _make_pass_a(vocab, emb, n_pad, e_chunks):
    n_chunks = n_pad // 128
    nck_w = n_chunks // (_NCORE * _NSUB)
    zck_s = n_pad // _NSUB // 128
    eck_w = e_chunks // (_NCORE * _NSUB)
    assert eck_w % 8 == 0
    eblk = eck_w // 8

    @functools.partial(
        pl.kernel,
        out_type=(
            jax.ShapeDtypeStruct((n_pad, emb), jnp.float32),
            jax.ShapeDtypeStruct((_NCORE * n_pad, 16), jnp.float32),
        ),
        mesh=_sc_mesh(),
        compiler_params=_SC_PARAMS,
        scratch_types=[
            pltpu.VMEM((nck_w, 128), jnp.int32),
            pltpu.VMEM((128, emb), jnp.float32),
            pltpu.VMEM((8, 128), jnp.int32),
            pltpu.VMEM((128, 16), jnp.float32),
            pltpu.VMEM((128, 16), jnp.float32),
            pltpu.VMEM_SHARED((n_pad, 16), jnp.float32),
            pltpu.SemaphoreType.DMA,
        ],
    )
    def pass_a(embed_hbm, x3d_hbm, dst2d_hbm, ones_hbm, zeros_hbm,
               h0_hbm, degp_hbm, xbuf, rowbuf, dstbuf, onesbuf, zbuf,
               acc, gsem):
        c = lax.axis_index("c")
        s = lax.axis_index("s")
        wid = c * _NSUB + s
        pltpu.sync_copy(ones_hbm, onesbuf)
        pltpu.sync_copy(zeros_hbm, zbuf)

        zbase = s * (n_pad // _NSUB)

        @pl.loop(0, zck_s)
        def _(k):
            pltpu.sync_copy(zbuf, acc.at[pl.ds(zbase + k * 128, 128)])

        nb = wid * nck_w
        pltpu.sync_copy(x3d_hbm.at[wid], xbuf)

        @pl.loop(0, nck_w)
        def _(k):
            pltpu.async_copy(embed_hbm.at[xbuf.at[k]], rowbuf, gsem).wait()
            pltpu.sync_copy(rowbuf, h0_hbm.at[pl.ds((nb + k) * 128, 128)])

        plsc.subcore_barrier()

        ebase = c * (e_chunks // _NCORE) + s * eck_w

        @pl.loop(0, eblk)
        def _(b):
            pltpu.sync_copy(dst2d_hbm.at[pl.ds(ebase + b * 8, 8)], dstbuf)
            for k in range(8):
                pltpu.sync_copy(onesbuf, acc.at[dstbuf.at[k]], add=True)

        plsc.subcore_barrier()

        @pl.loop(0, zck_s)
        def _(k):
            r = zbase + k * 128
            pltpu.sync_copy(acc.at[pl.ds(r, 128)],
                            degp_hbm.at[pl.ds(c * n_pad + r, 128)])

    return pass_a


def _make_pass_b(n_pad, e_chunks):
    zck_s = n_pad // _NSUB // 128
    eck_s = e_chunks // _NSUB
    assert eck_s % 8 == 0
    eblk = eck_s // 8

    @functools.partial(
        pl.kernel,
        out_type=jax.ShapeDtypeStruct((_NCORE * n_pad, 32), jnp.float32),
        mesh=_sc_mesh(),
        compiler_params=_SC_PARAMS,
        scratch_types=[
            pltpu.VMEM((8, 128), jnp.int32),
            pltpu.VMEM((8, 128), jnp.int32),
            pltpu.VMEM((4, 128, 32), jnp.float32),
            pltpu.VMEM_SHARED((n_pad, 32), jnp.float32),
            pltpu.SemaphoreType.DMA,
        ],
    )
    def pass_b(p_hbm, srcix_hbm, dst2d_hbm, zeros_hbm, agg_hbm,
               srcbuf, dstbuf, rows, acc, gsem):
        c = lax.axis_index("c")
        s = lax.axis_index("s")
        zbase = s * (n_pad // _NSUB)

        @pl.loop(0, zck_s)
        def _(k):
            pltpu.sync_copy(zeros_hbm, acc.at[pl.ds(zbase + k * 128, 128)])

        plsc.subcore_barrier()

        sbase = c * e_chunks + s * eck_s
        dbase = s * eck_s

        @pl.loop(0, eblk)
        def _(b):
            pltpu.sync_copy(srcix_hbm.at[pl.ds(sbase + b * 8, 8)], srcbuf)
            pltpu.sync_copy(dst2d_hbm.at[pl.ds(dbase + b * 8, 8)], dstbuf)
            for half in range(2):
                cps = []
                for k in range(4):
                    cps.append(pltpu.async_copy(
                        p_hbm.at[srcbuf.at[half * 4 + k]], rows.at[k], gsem))
                for k in range(4):
                    cps[k].wait()
                    pltpu.sync_copy(rows.at[k],
                                    acc.at[dstbuf.at[half * 4 + k]], add=True)

        plsc.subcore_barrier()

        @pl.loop(0, zck_s)
        def _(k):
            r = zbase + k * 128
            pltpu.sync_copy(acc.at[pl.ds(r, 128)],
                            agg_hbm.at[pl.ds(c * n_pad + r, 128)])

    return pass_b


def _dot_t(a, w):
    return lax.dot_general(a, w, (((1,), (1,)), ((), ())),
                           preferred_element_type=jnp.float32)


def _tc_layer_in(h0, wl, wr, bl, n_pad):
    nblk = n_pad // _BLK

    def body(h_ref, wl_ref, wr_ref, bl_ref, p_ref, s_ref):
        h = h_ref[...]
        p = _dot_t(h, wl_ref[...])
        p_ref[0] = p[:, :32]
        p_ref[1] = p[:, 32:]
        s_ref[...] = _dot_t(h, wr_ref[...]) + bl_ref[...]

    return pl.pallas_call(
        body,
        grid=(nblk,),
        in_specs=[
            pl.BlockSpec((_BLK, 64), lambda i: (i, 0)),
            pl.BlockSpec((64, 64), lambda i: (0, 0)),
            pl.BlockSpec((64, 64), lambda i: (0, 0)),
            pl.BlockSpec((1, 64), lambda i: (0, 0)),
        ],
        out_specs=[
            pl.BlockSpec((2, _BLK, 32), lambda i: (0, i, 0)),
            pl.BlockSpec((_BLK, 64), lambda i: (i, 0)),
        ],
        out_shape=[
            jax.ShapeDtypeStruct((2, n_pad, 32), jnp.float32),
            jax.ShapeDtypeStruct((n_pad, 64), jnp.float32),
        ],
    )(h0, wl, wr, bl.reshape(1, 64))


def _finish_layer(agg_ref, degp_ref, s_ref):
    a = jnp.concatenate([agg_ref[0], agg_ref[1]], axis=1)
    deg = degp_ref[0, :, 0:1] + degp_ref[1, :, 0:1]
    inv = 1.0 / jnp.maximum(deg, 1.0)
    return jax.nn.relu(a * inv + s_ref[...])


def _tc_layer_mid(agg3, degp3, s1, wl, wr, bl, n_pad):
    nblk = n_pad // _BLK

    def body(agg_ref, degp_ref, s_ref, wl_ref, wr_ref, bl_ref, p_ref, o_ref):
        h = _finish_layer(agg_ref, degp_ref, s_ref)
        p = _dot_t(h, wl_ref[...])
        p_ref[0] = p[:, :32]
        p_ref[1] = p[:, 32:]
        o_ref[...] = _dot_t(h, wr_ref[...]) + bl_ref[...]

    return pl.pallas_call(
        body,
        grid=(nblk,),
        in_specs=[
            pl.BlockSpec((2, _BLK, 32), lambda i: (0, i, 0)),
            pl.BlockSpec((2, _BLK, 16), lambda i: (0, i, 0)),
            pl.BlockSpec((_BLK, 64), lambda i: (i, 0)),
            pl.BlockSpec((64, 64), lambda i: (0, 0)),
            pl.BlockSpec((64, 64), lambda i: (0, 0)),
            pl.BlockSpec((1, 64), lambda i: (0, 0)),
        ],
        out_specs=[
            pl.BlockSpec((2, _BLK, 32), lambda i: (0, i, 0)),
            pl.BlockSpec((_BLK, 64), lambda i: (i, 0)),
        ],
        out_shape=[
            jax.ShapeDtypeStruct((2, n_pad, 32), jnp.float32),
            jax.ShapeDtypeStruct((n_pad, 64), jnp.float32),
        ],
    )(agg3, degp3, s1, wl, wr, bl.reshape(1, 64))


def _tc_pool(agg3, degp3, s2, batch3, wlin_p, blin_p, n_pad, ng, ncp):
    nblk = n_pad // _BLK

    def body(agg_ref, degp_ref, s_ref, b_ref, wlin_ref, blin_ref, o_ref,
             gsum, cnt):
        i = pl.program_id(0)
        h = _finish_layer(agg_ref, degp_ref, s_ref)
        b = b_ref[0]
        oh = (lax.broadcasted_iota(jnp.int32, (ng, _BLK), 0) == b
              ).astype(jnp.float32)

        @pl.when(i == 0)
        def _():
            gsum[...] = jnp.zeros_like(gsum)
            cnt[...] = jnp.zeros_like(cnt)

        gsum[...] += lax.dot_general(oh, h, (((1,), (0,)), ((), ())),
                                     preferred_element_type=jnp.float32)
        cnt[...] += jnp.sum(oh, axis=1, keepdims=True)

        @pl.when(i == nblk - 1)
        def _():
            g = gsum[...] / jnp.maximum(cnt[...], 1.0)
            o_ref[...] = _dot_t(g, wlin_ref[...]) + blin_ref[...]

    return pl.pallas_call(
        body,
        grid=(nblk,),
        in_specs=[
            pl.BlockSpec((2, _BLK, 32), lambda i: (0, i, 0)),
            pl.BlockSpec((2, _BLK, 16), lambda i: (0, i, 0)),
            pl.BlockSpec((_BLK, 64), lambda i: (i, 0)),
            pl.BlockSpec((1, 1, _BLK), lambda i: (i, 0, 0)),
            pl.BlockSpec((ncp, 64), lambda i: (0, 0)),
            pl.BlockSpec((1, ncp), lambda i: (0, 0)),
        ],
        out_specs=pl.BlockSpec((ng, ncp), lambda i: (0, 0)),
        out_shape=jax.ShapeDtypeStruct((ng, ncp), jnp.float32),
        scratch_shapes=[
            pltpu.VMEM((ng, 64), jnp.float32),
            pltpu.VMEM((ng, 1), jnp.float32),
        ],
    )(agg3, degp3, s2, batch3, wlin_p, blin_p)


def _ceil_to(v, m):
    return (v + m - 1) // m * m


def kernel(x, edge_index, batch, embed, Wl1, bl1, Wr1, Wl2, bl2, Wr2,
           Wlin, blin):
    n = x.shape[0]
    e = edge_index.shape[1]
    vocab, emb = embed.shape
    nc = Wlin.shape[0]
    ng = 128

    n_pad = _ceil_to(n, 128 * _NCORE * _NSUB)
    e_pad = _ceil_to(e, 128 * 8 * _NCORE * _NSUB)
    n_chunks = n_pad // 128
    e_chunks = e_pad // 128
    assert n_pad % _BLK == 0

    x_p = jnp.concatenate([x, jnp.zeros((n_pad - n,), jnp.int32)])
    x3d = x_p.reshape(_NCORE * _NSUB, n_chunks // (_NCORE * _NSUB), 128)
    src = edge_index[0]
    dst = edge_index[1]
    src_p = jnp.concatenate([src, jnp.zeros((e_pad - e,), jnp.int32)])
    dst_p = jnp.concatenate([dst, jnp.full((e_pad - e,), n, jnp.int32)])
    src2d = src_p.reshape(e_chunks, 128)
    srcix = jnp.concatenate([src2d, src2d + n_pad], axis=0)
    dst2d = dst_p.reshape(e_chunks, 128)
    batch_p = jnp.concatenate([batch, jnp.full((n_pad - n,), ng, jnp.int32)])
    batch3 = batch_p.reshape(n_pad // _BLK, 1, _BLK)

    ones16 = jnp.ones((128, 16), jnp.float32)
    zeros16 = jnp.zeros((128, 16), jnp.float32)
    zeros32 = jnp.zeros((128, 32), jnp.float32)

    ncp = _ceil_to(nc, 16)
    wlin_p = jnp.concatenate([Wlin, jnp.zeros((ncp - nc, 64), jnp.float32)])
    blin_p = jnp.concatenate([blin, jnp.zeros((ncp - nc,), jnp.float32)])
    blin_p = blin_p.reshape(1, ncp)

    pass_a = _make_pass_a(vocab, emb, n_pad, e_chunks)
    h0, degp = pass_a(embed, x3d, dst2d, ones16, zeros16)
    degp3 = degp.reshape(_NCORE, n_pad, 16)

    pass_b = _make_pass_b(n_pad, e_chunks)

    p1, s1 = _tc_layer_in(h0, Wl1, Wr1, bl1, n_pad)
    agg1 = pass_b(p1.reshape(_NCORE * n_pad, 32), srcix, dst2d, zeros32)
    agg1_3 = agg1.reshape(_NCORE, n_pad, 32)

    p2, s2 = _tc_layer_mid(agg1_3, degp3, s1, Wl2, Wr2, bl2, n_pad)
    agg2 = pass_b(p2.reshape(_NCORE * n_pad, 32), srcix, dst2d, zeros32)
    agg2_3 = agg2.reshape(_NCORE, n_pad, 32)

    out_p = _tc_pool(agg2_3, degp3, s2, batch3, wlin_p, blin_p, n_pad, ng, ncp)
    return out_p[:, :nc]


# speedup vs baseline: 7.7422x; 1.5694x over previous
import functools

import jax
import jax.numpy as jnp
from jax import lax
from jax.experimental import pallas as pl
from jax.experimental.pallas import tpu as pltpu
from jax.experimental.pallas import tpu_sc as plsc

_NCORE = 2
_NSUB = 16
_BLK = 1024


def _sc_mesh():
    return plsc.VectorSubcoreMesh(core_axis_name="c", subcore_axis_name="s")


_SC_PARAMS = pltpu.CompilerParams(use_tc_tiling_on_sc=False)


def _make_pass_a(vocab, emb, n_pad, e_chunks):
    n_chunks = n_pad // 128
    nck_w = n_chunks // (_NCORE * _NSUB)
    zck_s = n_pad // _NSUB // 128
    eck_w = e_chunks // (_NCORE * _NSUB)
    assert eck_w % 8 == 0
    eblk = eck_w // 8

    @functools.partial(
        pl.kernel,
        out_type=(
            jax.ShapeDtypeStruct((n_pad, emb), jnp.float32),
            jax.ShapeDtypeStruct((_NCORE, n_pad, 16), jnp.float32),
        ),
        mesh=_sc_mesh(),
        compiler_params=_SC_PARAMS,
        scratch_types=[
            pltpu.VMEM((nck_w, 128), jnp.int32),
            pltpu.VMEM((128, emb), jnp.float32),
            pltpu.VMEM((8, 128), jnp.int32),
            pltpu.VMEM((128, 16), jnp.float32),
            pltpu.VMEM((128, 16), jnp.float32),
            pltpu.VMEM_SHARED((n_pad, 16), jnp.float32),
            pltpu.SemaphoreType.DMA,
        ],
    )
    def pass_a(embed_hbm, x3d_hbm, dst2d_hbm, ones_hbm, zeros_hbm,
               h0_hbm, degp_hbm, xbuf, rowbuf, dstbuf, onesbuf, zbuf,
               acc, gsem):
        c = lax.axis_index("c")
        s = lax.axis_index("s")
        wid = c * _NSUB + s
        pltpu.sync_copy(ones_hbm, onesbuf)
        pltpu.sync_copy(zeros_hbm, zbuf)

        zbase = s * (n_pad // _NSUB)

        @pl.loop(0, zck_s)
        def _(k):
            pltpu.sync_copy(zbuf, acc.at[pl.ds(zbase + k * 128, 128)])

        nb = wid * nck_w
        pltpu.sync_copy(x3d_hbm.at[wid], xbuf)

        @pl.loop(0, nck_w)
        def _(k):
            pltpu.async_copy(embed_hbm.at[xbuf.at[k]], rowbuf, gsem).wait()
            pltpu.sync_copy(rowbuf, h0_hbm.at[pl.ds((nb + k) * 128, 128)])

        plsc.subcore_barrier()

        ebase = c * (e_chunks // _NCORE) + s * eck_w

        @pl.loop(0, eblk)
        def _(b):
            pltpu.sync_copy(dst2d_hbm.at[pl.ds(ebase + b * 8, 8)], dstbuf)
            for k in range(8):
                pltpu.sync_copy(onesbuf, acc.at[dstbuf.at[k]], add=True)

        plsc.subcore_barrier()

        @pl.loop(0, zck_s)
        def _(k):
            r = zbase + k * 128
            pltpu.sync_copy(acc.at[pl.ds(r, 128)],
                            degp_hbm.at[c].at[pl.ds(r, 128)])

    return pass_a


def _make_pass_b(n_pad, e_chunks):
    zck_s = n_pad // _NSUB // 128
    eck_s = e_chunks // _NSUB
    assert eck_s % 8 == 0
    eblk = eck_s // 8

    @functools.partial(
        pl.kernel,
        out_type=jax.ShapeDtypeStruct((_NCORE, n_pad, 32), jnp.float32),
        mesh=_sc_mesh(),
        compiler_params=_SC_PARAMS,
        scratch_types=[
            pltpu.VMEM((8, 128), jnp.int32),
            pltpu.VMEM((8, 128), jnp.int32),
            pltpu.VMEM((4, 128, 32), jnp.float32),
            pltpu.VMEM_SHARED((n_pad, 32), jnp.float32),
            pltpu.SemaphoreType.DMA,
        ],
    )
    def pass_b(p_hbm, src2d_hbm, dst2d_hbm, zeros_hbm, agg_hbm,
               srcbuf, dstbuf, rows, acc, gsem):
        c = lax.axis_index("c")
        s = lax.axis_index("s")
        zbase = s * (n_pad // _NSUB)

        @pl.loop(0, zck_s)
        def _(k):
            pltpu.sync_copy(zeros_hbm, acc.at[pl.ds(zbase + k * 128, 128)])

        plsc.subcore_barrier()

        base = s * eck_s

        @pl.loop(0, eblk)
        def _(b):
            pltpu.sync_copy(src2d_hbm.at[pl.ds(base + b * 8, 8)], srcbuf)
            pltpu.sync_copy(dst2d_hbm.at[pl.ds(base + b * 8, 8)], dstbuf)
            for half in range(2):
                cps = []
                for k in range(4):
                    cps.append(pltpu.async_copy(
                        p_hbm.at[c].at[srcbuf.at[half * 4 + k]],
                        rows.at[k], gsem))
                for k in range(4):
                    cps[k].wait()
                    pltpu.sync_copy(rows.at[k],
                                    acc.at[dstbuf.at[half * 4 + k]], add=True)

        plsc.subcore_barrier()

        @pl.loop(0, zck_s)
        def _(k):
            r = zbase + k * 128
            pltpu.sync_copy(acc.at[pl.ds(r, 128)],
                            agg_hbm.at[c].at[pl.ds(r, 128)])

    return pass_b


def _dot_t(a, w):
    return lax.dot_general(a, w, (((1,), (1,)), ((), ())),
                           preferred_element_type=jnp.float32)


def _tc_layer_in(h0, wl, wr, bl, n_pad):
    nblk = n_pad // _BLK

    def body(h_ref, wl_ref, wr_ref, bl_ref, p_ref, s_ref):
        h = h_ref[...]
        p = _dot_t(h, wl_ref[...])
        p_ref[0] = p[:, :32]
        p_ref[1] = p[:, 32:]
        s_ref[...] = _dot_t(h, wr_ref[...]) + bl_ref[...]

    return pl.pallas_call(
        body,
        grid=(nblk,),
        in_specs=[
            pl.BlockSpec((_BLK, 64), lambda i: (i, 0)),
            pl.BlockSpec((64, 64), lambda i: (0, 0)),
            pl.BlockSpec((64, 64), lambda i: (0, 0)),
            pl.BlockSpec((1, 64), lambda i: (0, 0)),
        ],
        out_specs=[
            pl.BlockSpec((2, _BLK, 32), lambda i: (0, i, 0)),
            pl.BlockSpec((_BLK, 64), lambda i: (i, 0)),
        ],
        out_shape=[
            jax.ShapeDtypeStruct((2, n_pad, 32), jnp.float32),
            jax.ShapeDtypeStruct((n_pad, 64), jnp.float32),
        ],
    )(h0, wl, wr, bl.reshape(1, 64))


def _finish_layer(agg_ref, degp_ref, s_ref):
    a = jnp.concatenate([agg_ref[0], agg_ref[1]], axis=1)
    deg = degp_ref[0, :, 0:1] + degp_ref[1, :, 0:1]
    inv = 1.0 / jnp.maximum(deg, 1.0)
    return jax.nn.relu(a * inv + s_ref[...])


def _tc_layer_mid(agg3, degp3, s1, wl, wr, bl, n_pad):
    nblk = n_pad // _BLK

    def body(agg_ref, degp_ref, s_ref, wl_ref, wr_ref, bl_ref, p_ref, o_ref):
        h = _finish_layer(agg_ref, degp_ref, s_ref)
        p = _dot_t(h, wl_ref[...])
        p_ref[0] = p[:, :32]
        p_ref[1] = p[:, 32:]
        o_ref[...] = _dot_t(h, wr_ref[...]) + bl_ref[...]

    return pl.pallas_call(
        body,
        grid=(nblk,),
        in_specs=[
            pl.BlockSpec((2, _BLK, 32), lambda i: (0, i, 0)),
            pl.BlockSpec((2, _BLK, 16), lambda i: (0, i, 0)),
            pl.BlockSpec((_BLK, 64), lambda i: (i, 0)),
            pl.BlockSpec((64, 64), lambda i: (0, 0)),
            pl.BlockSpec((64, 64), lambda i: (0, 0)),
            pl.BlockSpec((1, 64), lambda i: (0, 0)),
        ],
        out_specs=[
            pl.BlockSpec((2, _BLK, 32), lambda i: (0, i, 0)),
            pl.BlockSpec((_BLK, 64), lambda i: (i, 0)),
        ],
        out_shape=[
            jax.ShapeDtypeStruct((2, n_pad, 32), jnp.float32),
            jax.ShapeDtypeStruct((n_pad, 64), jnp.float32),
        ],
    )(agg3, degp3, s1, wl, wr, bl.reshape(1, 64))


def _tc_pool(agg3, degp3, s2, batch3, wlin_p, blin_p, n_pad, ng, ncp):
    nblk = n_pad // _BLK

    def body(agg_ref, degp_ref, s_ref, b_ref, wlin_ref, blin_ref, o_ref,
             gsum, cnt):
        i = pl.program_id(0)
        h = _finish_layer(agg_ref, degp_ref, s_ref)
        b = b_ref[0]
        oh = (lax.broadcasted_iota(jnp.int32, (ng, _BLK), 0) == b
              ).astype(jnp.float32)

        @pl.when(i == 0)
        def _():
            gsum[...] = jnp.zeros_like(gsum)
            cnt[...] = jnp.zeros_like(cnt)

        gsum[...] += lax.dot_general(oh, h, (((1,), (0,)), ((), ())),
                                     preferred_element_type=jnp.float32)
        cnt[...] += jnp.sum(oh, axis=1, keepdims=True)

        @pl.when(i == nblk - 1)
        def _():
            g = gsum[...] / jnp.maximum(cnt[...], 1.0)
            o_ref[...] = _dot_t(g, wlin_ref[...]) + blin_ref[...]

    return pl.pallas_call(
        body,
        grid=(nblk,),
        in_specs=[
            pl.BlockSpec((2, _BLK, 32), lambda i: (0, i, 0)),
            pl.BlockSpec((2, _BLK, 16), lambda i: (0, i, 0)),
            pl.BlockSpec((_BLK, 64), lambda i: (i, 0)),
            pl.BlockSpec((1, 1, _BLK), lambda i: (i, 0, 0)),
            pl.BlockSpec((ncp, 64), lambda i: (0, 0)),
            pl.BlockSpec((1, ncp), lambda i: (0, 0)),
        ],
        out_specs=pl.BlockSpec((ng, ncp), lambda i: (0, 0)),
        out_shape=jax.ShapeDtypeStruct((ng, ncp), jnp.float32),
        scratch_shapes=[
            pltpu.VMEM((ng, 64), jnp.float32),
            pltpu.VMEM((ng, 1), jnp.float32),
        ],
    )(agg3, degp3, s2, batch3, wlin_p, blin_p)


def _ceil_to(v, m):
    return (v + m - 1) // m * m


def kernel(x, edge_index, batch, embed, Wl1, bl1, Wr1, Wl2, bl2, Wr2,
           Wlin, blin):
    n = x.shape[0]
    e = edge_index.shape[1]
    vocab, emb = embed.shape
    nc = Wlin.shape[0]
    ng = 128

    n_pad = _ceil_to(n, 128 * _NCORE * _NSUB)
    e_pad = _ceil_to(e, 128 * 8 * _NCORE * _NSUB)
    n_chunks = n_pad // 128
    e_chunks = e_pad // 128
    assert n_pad % _BLK == 0

    x_p = jnp.concatenate(
        [x, jnp.arange(n_pad - n, dtype=jnp.int32) % vocab])
    x3d = x_p.reshape(_NCORE * _NSUB, n_chunks // (_NCORE * _NSUB), 128)
    src = edge_index[0]
    dst = edge_index[1]
    src_p = jnp.concatenate(
        [src, jnp.arange(e_pad - e, dtype=jnp.int32) % n])
    dst_p = jnp.concatenate(
        [dst, n + jnp.arange(e_pad - e, dtype=jnp.int32) % (n_pad - n)])
    src2d = src_p.reshape(e_chunks, 128)
    dst2d = dst_p.reshape(e_chunks, 128)
    batch_p = jnp.concatenate([batch, jnp.full((n_pad - n,), ng, jnp.int32)])
    batch3 = batch_p.reshape(n_pad // _BLK, 1, _BLK)

    ones16 = jnp.ones((128, 16), jnp.float32)
    zeros16 = jnp.zeros((128, 16), jnp.float32)
    zeros32 = jnp.zeros((128, 32), jnp.float32)

    ncp = _ceil_to(nc, 16)
    wlin_p = jnp.concatenate([Wlin, jnp.zeros((ncp - nc, 64), jnp.float32)])
    blin_p = jnp.concatenate([blin, jnp.zeros((ncp - nc,), jnp.float32)])
    blin_p = blin_p.reshape(1, ncp)

    pass_a = _make_pass_a(vocab, emb, n_pad, e_chunks)
    h0, degp3 = pass_a(embed, x3d, dst2d, ones16, zeros16)

    pass_b = _make_pass_b(n_pad, e_chunks)

    p1, s1 = _tc_layer_in(h0, Wl1, Wr1, bl1, n_pad)
    agg1_3 = pass_b(p1, src2d, dst2d, zeros32)

    p2, s2 = _tc_layer_mid(agg1_3, degp3, s1, Wl2, Wr2, bl2, n_pad)
    agg2_3 = pass_b(p2, src2d, dst2d, zeros32)

    out_p = _tc_pool(agg2_3, degp3, s2, batch3, wlin_p, blin_p, n_pad, ng, ncp)
    return out_p[:, :nc]


# speedup vs baseline: 10.3488x; 1.3367x over previous
import functools

import jax
import jax.numpy as jnp
from jax import lax
from jax.experimental import pallas as pl
from jax.experimental.pallas import tpu as pltpu
from jax.experimental.pallas import tpu_sc as plsc

_NCORE = 2
_NSUB = 16
_BLK = 4096


def _sc_mesh():
    return plsc.VectorSubcoreMesh(core_axis_name="c", subcore_axis_name="s")


_SC_PARAMS = pltpu.CompilerParams(use_tc_tiling_on_sc=False)


def _make_pass_a(vocab, emb, n_pad, e_chunks):
    n_chunks = n_pad // 128
    nck_w = n_chunks // (_NCORE * _NSUB)
    zck_s = n_pad // _NSUB // 128
    eck_w = e_chunks // (_NCORE * _NSUB)
    assert eck_w % 8 == 0
    eblk = eck_w // 8

    @functools.partial(
        pl.kernel,
        out_type=(
            jax.ShapeDtypeStruct((n_pad, emb), jnp.float32),
            jax.ShapeDtypeStruct((_NCORE, n_pad, 16), jnp.float32),
        ),
        mesh=_sc_mesh(),
        compiler_params=_SC_PARAMS,
        scratch_types=[
            pltpu.VMEM((nck_w, 128), jnp.int32),
            pltpu.VMEM((128, emb), jnp.float32),
            pltpu.VMEM((8, 128), jnp.int32),
            pltpu.VMEM((128, 16), jnp.float32),
            pltpu.VMEM((128, 16), jnp.float32),
            pltpu.VMEM_SHARED((n_pad, 16), jnp.float32),
            pltpu.SemaphoreType.DMA,
        ],
    )
    def pass_a(embed_hbm, x3d_hbm, dst2d_hbm, ones_hbm, zeros_hbm,
               h0_hbm, degp_hbm, xbuf, rowbuf, dstbuf, onesbuf, zbuf,
               acc, gsem):
        c = lax.axis_index("c")
        s = lax.axis_index("s")
        wid = c * _NSUB + s
        pltpu.sync_copy(ones_hbm, onesbuf)
        pltpu.sync_copy(zeros_hbm, zbuf)

        zbase = s * (n_pad // _NSUB)

        @pl.loop(0, zck_s)
        def _(k):
            pltpu.sync_copy(zbuf, acc.at[pl.ds(zbase + k * 128, 128)])

        nb = wid * nck_w
        pltpu.sync_copy(x3d_hbm.at[wid], xbuf)

        @pl.loop(0, nck_w)
        def _(k):
            pltpu.async_copy(embed_hbm.at[xbuf.at[k]], rowbuf, gsem).wait()
            pltpu.sync_copy(rowbuf, h0_hbm.at[pl.ds((nb + k) * 128, 128)])

        plsc.subcore_barrier()

        ebase = c * (e_chunks // _NCORE) + s * eck_w

        @pl.loop(0, eblk)
        def _(b):
            pltpu.sync_copy(dst2d_hbm.at[pl.ds(ebase + b * 8, 8)], dstbuf)
            for k in range(8):
                pltpu.sync_copy(onesbuf, acc.at[dstbuf.at[k]], add=True)

        plsc.subcore_barrier()

        @pl.loop(0, zck_s)
        def _(k):
            r = zbase + k * 128
            pltpu.sync_copy(acc.at[pl.ds(r, 128)],
                            degp_hbm.at[c].at[pl.ds(r, 128)])

    return pass_a


def _make_pass_b(n_pad, e_chunks):
    zck_s = n_pad // _NSUB // 128
    eck_s = e_chunks // _NSUB
    assert eck_s % 4 == 0
    nb = eck_s // 4

    @functools.partial(
        pl.kernel,
        out_type=jax.ShapeDtypeStruct((_NCORE, n_pad, 32), jnp.float32),
        mesh=_sc_mesh(),
        compiler_params=_SC_PARAMS,
        scratch_types=[
            pltpu.VMEM((3, 4, 128), jnp.int32),
            pltpu.VMEM((3, 4, 128), jnp.int32),
            pltpu.VMEM((4, 128, 32), jnp.float32),
            pltpu.VMEM_SHARED((n_pad, 32), jnp.float32),
            pltpu.SemaphoreType.DMA,
            pltpu.SemaphoreType.DMA,
        ],
    )
    def pass_b(p_hbm, src2d_hbm, dst2d_hbm, zeros_hbm, agg_hbm,
               srcbuf, dstbuf, rows, acc, gsem, isem):
        c = lax.axis_index("c")
        s = lax.axis_index("s")
        zbase = s * (n_pad // _NSUB)

        @pl.loop(0, zck_s)
        def _(k):
            pltpu.sync_copy(zeros_hbm, acc.at[pl.ds(zbase + k * 128, 128)])

        plsc.subcore_barrier()

        base = s * eck_s

        def load_idx_async(b, slot):
            rowa = base + b * 4
            pltpu.async_copy(src2d_hbm.at[pl.ds(rowa, 4)],
                             srcbuf.at[slot], isem)
            pltpu.async_copy(dst2d_hbm.at[pl.ds(rowa, 4)],
                             dstbuf.at[slot], isem)

        def drain_idx(slot):
            pltpu.make_async_copy(src2d_hbm.at[pl.ds(0, 4)],
                                  srcbuf.at[slot], isem).wait()
            pltpu.make_async_copy(dst2d_hbm.at[pl.ds(0, 4)],
                                  dstbuf.at[slot], isem).wait()

        def issue_gather(slot, k):
            pltpu.async_copy(p_hbm.at[c].at[srcbuf.at[slot, k]],
                             rows.at[k], gsem)

        def wait_gather(k):
            pltpu.make_async_copy(p_hbm.at[c].at[pl.ds(0, 128)],
                                  rows.at[k], gsem).wait()

        def scatter(slot, k):
            pltpu.sync_copy(rows.at[k], acc.at[dstbuf.at[slot, k]], add=True)

        pltpu.sync_copy(src2d_hbm.at[pl.ds(base, 4)], srcbuf.at[0])
        pltpu.sync_copy(dst2d_hbm.at[pl.ds(base, 4)], dstbuf.at[0])
        load_idx_async(1, 1)
        for k in range(4):
            issue_gather(0, k)

        @pl.loop(0, nb - 2)
        def _(b):
            cur = lax.rem(b, 3)
            nxt = lax.rem(b + 1, 3)
            pre = lax.rem(b + 2, 3)
            drain_idx(nxt)
            load_idx_async(b + 2, pre)
            for k in range(4):
                wait_gather(k)
                scatter(cur, k)
                issue_gather(nxt, k)

        cur = (nb - 2) % 3
        nxt = (nb - 1) % 3
        drain_idx(nxt)
        for k in range(4):
            wait_gather(k)
            scatter(cur, k)
            issue_gather(nxt, k)

        cur = (nb - 1) % 3
        for k in range(4):
            wait_gather(k)
            scatter(cur, k)

        plsc.subcore_barrier()

        @pl.loop(0, zck_s)
        def _(k):
            r = zbase + k * 128
            pltpu.sync_copy(acc.at[pl.ds(r, 128)],
                            agg_hbm.at[c].at[pl.ds(r, 128)])

    return pass_b


def _dot_t(a, w):
    return lax.dot_general(a, w, (((1,), (1,)), ((), ())),
                           preferred_element_type=jnp.float32)


def _tc_layer_in(h0, wl, wr, bl, n_pad):
    nblk = n_pad // _BLK

    def body(h_ref, wl_ref, wr_ref, bl_ref, p_ref, s_ref):
        h = h_ref[...]
        p = _dot_t(h, wl_ref[...])
        p_ref[0] = p[:, :32]
        p_ref[1] = p[:, 32:]
        s_ref[...] = _dot_t(h, wr_ref[...]) + bl_ref[...]

    return pl.pallas_call(
        body,
        grid=(nblk,),
        in_specs=[
            pl.BlockSpec((_BLK, 64), lambda i: (i, 0)),
            pl.BlockSpec((64, 64), lambda i: (0, 0)),
            pl.BlockSpec((64, 64), lambda i: (0, 0)),
            pl.BlockSpec((1, 64), lambda i: (0, 0)),
        ],
        out_specs=[
            pl.BlockSpec((2, _BLK, 32), lambda i: (0, i, 0)),
            pl.BlockSpec((_BLK, 64), lambda i: (i, 0)),
        ],
        out_shape=[
            jax.ShapeDtypeStruct((2, n_pad, 32), jnp.float32),
            jax.ShapeDtypeStruct((n_pad, 64), jnp.float32),
        ],
    )(h0, wl, wr, bl.reshape(1, 64))


def _finish_layer(agg_ref, degp_ref, s_ref):
    a = jnp.concatenate([agg_ref[0], agg_ref[1]], axis=1)
    deg = degp_ref[0, :, 0:1] + degp_ref[1, :, 0:1]
    inv = 1.0 / jnp.maximum(deg, 1.0)
    return jax.nn.relu(a * inv + s_ref[...])


def _tc_layer_mid(agg3, degp3, s1, wl, wr, bl, n_pad):
    nblk = n_pad // _BLK

    def body(agg_ref, degp_ref, s_ref, wl_ref, wr_ref, bl_ref, p_ref, o_ref):
        h = _finish_layer(agg_ref, degp_ref, s_ref)
        p = _dot_t(h, wl_ref[...])
        p_ref[0] = p[:, :32]
        p_ref[1] = p[:, 32:]
        o_ref[...] = _dot_t(h, wr_ref[...]) + bl_ref[...]

    return pl.pallas_call(
        body,
        grid=(nblk,),
        in_specs=[
            pl.BlockSpec((2, _BLK, 32), lambda i: (0, i, 0)),
            pl.BlockSpec((2, _BLK, 16), lambda i: (0, i, 0)),
            pl.BlockSpec((_BLK, 64), lambda i: (i, 0)),
            pl.BlockSpec((64, 64), lambda i: (0, 0)),
            pl.BlockSpec((64, 64), lambda i: (0, 0)),
            pl.BlockSpec((1, 64), lambda i: (0, 0)),
        ],
        out_specs=[
            pl.BlockSpec((2, _BLK, 32), lambda i: (0, i, 0)),
            pl.BlockSpec((_BLK, 64), lambda i: (i, 0)),
        ],
        out_shape=[
            jax.ShapeDtypeStruct((2, n_pad, 32), jnp.float32),
            jax.ShapeDtypeStruct((n_pad, 64), jnp.float32),
        ],
    )(agg3, degp3, s1, wl, wr, bl.reshape(1, 64))


def _tc_pool(agg3, degp3, s2, batch3, wlin_p, blin_p, n_pad, ng, ncp):
    nblk = n_pad // _BLK

    def body(agg_ref, degp_ref, s_ref, b_ref, wlin_ref, blin_ref, o_ref,
             gsum, cnt):
        i = pl.program_id(0)
        h = _finish_layer(agg_ref, degp_ref, s_ref)
        b = b_ref[0]
        oh = (lax.broadcasted_iota(jnp.int32, (ng, _BLK), 0) == b
              ).astype(jnp.float32)

        @pl.when(i == 0)
        def _():
            gsum[...] = jnp.zeros_like(gsum)
            cnt[...] = jnp.zeros_like(cnt)

        gsum[...] += lax.dot_general(oh, h, (((1,), (0,)), ((), ())),
                                     preferred_element_type=jnp.float32)
        cnt[...] += jnp.sum(oh, axis=1, keepdims=True)

        @pl.when(i == nblk - 1)
        def _():
            g = gsum[...] / jnp.maximum(cnt[...], 1.0)
            o_ref[...] = _dot_t(g, wlin_ref[...]) + blin_ref[...]

    return pl.pallas_call(
        body,
        grid=(nblk,),
        in_specs=[
            pl.BlockSpec((2, _BLK, 32), lambda i: (0, i, 0)),
            pl.BlockSpec((2, _BLK, 16), lambda i: (0, i, 0)),
            pl.BlockSpec((_BLK, 64), lambda i: (i, 0)),
            pl.BlockSpec((1, 1, _BLK), lambda i: (i, 0, 0)),
            pl.BlockSpec((ncp, 64), lambda i: (0, 0)),
            pl.BlockSpec((1, ncp), lambda i: (0, 0)),
        ],
        out_specs=pl.BlockSpec((ng, ncp), lambda i: (0, 0)),
        out_shape=jax.ShapeDtypeStruct((ng, ncp), jnp.float32),
        scratch_shapes=[
            pltpu.VMEM((ng, 64), jnp.float32),
            pltpu.VMEM((ng, 1), jnp.float32),
        ],
    )(agg3, degp3, s2, batch3, wlin_p, blin_p)


def _ceil_to(v, m):
    return (v + m - 1) // m * m


def kernel(x, edge_index, batch, embed, Wl1, bl1, Wr1, Wl2, bl2, Wr2,
           Wlin, blin):
    n = x.shape[0]
    e = edge_index.shape[1]
    vocab, emb = embed.shape
    nc = Wlin.shape[0]
    ng = 128

    n_pad = _ceil_to(n, 128 * _NCORE * _NSUB)
    e_pad = _ceil_to(e, 128 * 8 * _NCORE * _NSUB)
    n_chunks = n_pad // 128
    e_chunks = e_pad // 128
    assert n_pad % _BLK == 0

    x_p = jnp.concatenate(
        [x, jnp.arange(n_pad - n, dtype=jnp.int32) % vocab])
    x3d = x_p.reshape(_NCORE * _NSUB, n_chunks // (_NCORE * _NSUB), 128)
    src = edge_index[0]
    dst = edge_index[1]
    src_p = jnp.concatenate(
        [src, jnp.arange(e_pad - e, dtype=jnp.int32) % n])
    dst_p = jnp.concatenate(
        [dst, n + jnp.arange(e_pad - e, dtype=jnp.int32) % (n_pad - n)])
    src2d = src_p.reshape(e_chunks, 128)
    dst2d = dst_p.reshape(e_chunks, 128)
    batch_p = jnp.concatenate([batch, jnp.full((n_pad - n,), ng, jnp.int32)])
    batch3 = batch_p.reshape(n_pad // _BLK, 1, _BLK)

    ones16 = jnp.ones((128, 16), jnp.float32)
    zeros16 = jnp.zeros((128, 16), jnp.float32)
    zeros32 = jnp.zeros((128, 32), jnp.float32)

    ncp = _ceil_to(nc, 16)
    wlin_p = jnp.concatenate([Wlin, jnp.zeros((ncp - nc, 64), jnp.float32)])
    blin_p = jnp.concatenate([blin, jnp.zeros((ncp - nc,), jnp.float32)])
    blin_p = blin_p.reshape(1, ncp)

    pass_a = _make_pass_a(vocab, emb, n_pad, e_chunks)
    h0, degp3 = pass_a(embed, x3d, dst2d, ones16, zeros16)

    pass_b = _make_pass_b(n_pad, e_chunks)

    p1, s1 = _tc_layer_in(h0, Wl1, Wr1, bl1, n_pad)
    agg1_3 = pass_b(p1, src2d, dst2d, zeros32)

    p2, s2 = _tc_layer_mid(agg1_3, degp3, s1, Wl2, Wr2, bl2, n_pad)
    agg2_3 = pass_b(p2, src2d, dst2d, zeros32)

    out_p = _tc_pool(agg2_3, degp3, s2, batch3, wlin_p, blin_p, n_pad, ng, ncp)
    return out_p[:, :nc]


# speedup vs baseline: 10.9003x; 1.0533x over previous
import functools

import jax
import jax.numpy as jnp
from jax import lax
from jax.experimental import pallas as pl
from jax.experimental.pallas import tpu as pltpu
from jax.experimental.pallas import tpu_sc as plsc

_NCORE = 2
_NSUB = 16
_BLK = 4096


def _sc_mesh():
    return plsc.VectorSubcoreMesh(core_axis_name="c", subcore_axis_name="s")


_SC_PARAMS = pltpu.CompilerParams(use_tc_tiling_on_sc=False)


def _make_pass_a(vocab, emb, n_pad, e_chunks):
    n_chunks = n_pad // 128
    nck_w = n_chunks // (_NCORE * _NSUB)
    zck_s = n_pad // _NSUB // 128
    eck_w = e_chunks // (_NCORE * _NSUB)
    assert eck_w % 8 == 0
    eblk = eck_w // 8

    @functools.partial(
        pl.kernel,
        out_type=(
            jax.ShapeDtypeStruct((n_pad, emb), jnp.float32),
            jax.ShapeDtypeStruct((_NCORE, n_pad, 16), jnp.float32),
        ),
        mesh=_sc_mesh(),
        compiler_params=_SC_PARAMS,
        scratch_types=[
            pltpu.VMEM((nck_w, 128), jnp.int32),
            pltpu.VMEM((4, 128, emb), jnp.float32),
            pltpu.VMEM((3, 8, 128), jnp.int32),
            pltpu.VMEM((128, 16), jnp.float32),
            pltpu.VMEM((128, 16), jnp.float32),
            pltpu.VMEM_SHARED((n_pad, 16), jnp.float32),
            pltpu.SemaphoreType.DMA,
            pltpu.SemaphoreType.DMA,
        ],
    )
    def pass_a(embed_hbm, x3d_hbm, dst2d_hbm, ones_hbm, zeros_hbm,
               h0_hbm, degp_hbm, xbuf, rows, dstbuf, onesbuf, zbuf,
               acc, gsem, isem):
        c = lax.axis_index("c")
        s = lax.axis_index("s")
        wid = c * _NSUB + s
        pltpu.sync_copy(ones_hbm, onesbuf)
        pltpu.sync_copy(zeros_hbm, zbuf)

        zbase = s * (n_pad // _NSUB)

        @pl.loop(0, zck_s)
        def _(k):
            pltpu.sync_copy(zbuf, acc.at[pl.ds(zbase + k * 128, 128)])

        nb = wid * nck_w
        pltpu.sync_copy(x3d_hbm.at[wid], xbuf)
        for k in range(min(4, nck_w)):
            pltpu.async_copy(embed_hbm.at[xbuf.at[k]], rows.at[k % 4], gsem)
        for k in range(nck_w):
            pltpu.make_async_copy(embed_hbm.at[pl.ds(0, 128)],
                                  rows.at[k % 4], gsem).wait()
            pltpu.sync_copy(rows.at[k % 4],
                            h0_hbm.at[pl.ds((nb + k) * 128, 128)])
            if k + 4 < nck_w:
                pltpu.async_copy(embed_hbm.at[xbuf.at[k + 4]],
                                 rows.at[k % 4], gsem)

        plsc.subcore_barrier()

        ebase = c * (e_chunks // _NCORE) + s * eck_w

        def load_idx_async(b, slot):
            pltpu.async_copy(dst2d_hbm.at[pl.ds(ebase + b * 8, 8)],
                             dstbuf.at[slot], isem)

        def drain_idx(slot):
            pltpu.make_async_copy(dst2d_hbm.at[pl.ds(0, 8)],
                                  dstbuf.at[slot], isem).wait()

        pltpu.sync_copy(dst2d_hbm.at[pl.ds(ebase, 8)], dstbuf.at[0])
        load_idx_async(1, 1)

        @pl.loop(0, eblk - 2)
        def _(b):
            cur = lax.rem(b, 3)
            drain_idx(lax.rem(b + 1, 3))
            load_idx_async(b + 2, lax.rem(b + 2, 3))
            for k in range(8):
                pltpu.sync_copy(onesbuf, acc.at[dstbuf.at[cur, k]], add=True)

        drain_idx((eblk - 1) % 3)
        for b in (eblk - 2, eblk - 1):
            for k in range(8):
                pltpu.sync_copy(onesbuf, acc.at[dstbuf.at[b % 3, k]],
                                add=True)

        plsc.subcore_barrier()

        @pl.loop(0, zck_s)
        def _(k):
            r = zbase + k * 128
            pltpu.sync_copy(acc.at[pl.ds(r, 128)],
                            degp_hbm.at[c].at[pl.ds(r, 128)])

    return pass_a


def _make_pass_b(n_pad, e_chunks):
    zck_s = n_pad // _NSUB // 128
    eck_s = e_chunks // _NSUB
    assert eck_s % 8 == 0
    nb = eck_s // 8

    @functools.partial(
        pl.kernel,
        out_type=jax.ShapeDtypeStruct((_NCORE, n_pad, 32), jnp.bfloat16),
        mesh=_sc_mesh(),
        compiler_params=_SC_PARAMS,
        scratch_types=[
            pltpu.VMEM((3, 8, 128), jnp.int32),
            pltpu.VMEM((3, 8, 128), jnp.int32),
            pltpu.VMEM((8, 128, 32), jnp.bfloat16),
            pltpu.VMEM_SHARED((n_pad, 32), jnp.bfloat16),
            pltpu.SemaphoreType.DMA,
            pltpu.SemaphoreType.DMA,
        ],
    )
    def pass_b(p_hbm, src2d_hbm, dst2d_hbm, zeros_hbm, agg_hbm,
               srcbuf, dstbuf, rows, acc, gsem, isem):
        c = lax.axis_index("c")
        s = lax.axis_index("s")
        zbase = s * (n_pad // _NSUB)

        @pl.loop(0, zck_s)
        def _(k):
            pltpu.sync_copy(zeros_hbm, acc.at[pl.ds(zbase + k * 128, 128)])

        plsc.subcore_barrier()

        base = s * eck_s

        def load_idx_async(b, slot):
            rowa = base + b * 8
            pltpu.async_copy(src2d_hbm.at[pl.ds(rowa, 8)],
                             srcbuf.at[slot], isem)
            pltpu.async_copy(dst2d_hbm.at[pl.ds(rowa, 8)],
                             dstbuf.at[slot], isem)

        def drain_idx(slot):
            pltpu.make_async_copy(src2d_hbm.at[pl.ds(0, 8)],
                                  srcbuf.at[slot], isem).wait()
            pltpu.make_async_copy(dst2d_hbm.at[pl.ds(0, 8)],
                                  dstbuf.at[slot], isem).wait()

        def issue_gather(slot, k):
            pltpu.async_copy(p_hbm.at[c].at[srcbuf.at[slot, k]],
                             rows.at[k], gsem)

        def wait_gather(k):
            pltpu.make_async_copy(p_hbm.at[c].at[pl.ds(0, 128)],
                                  rows.at[k], gsem).wait()

        def scatter(slot, k):
            pltpu.sync_copy(rows.at[k], acc.at[dstbuf.at[slot, k]], add=True)

        pltpu.sync_copy(src2d_hbm.at[pl.ds(base, 8)], srcbuf.at[0])
        pltpu.sync_copy(dst2d_hbm.at[pl.ds(base, 8)], dstbuf.at[0])
        load_idx_async(1, 1)
        for k in range(8):
            issue_gather(0, k)

        @pl.loop(0, nb - 2)
        def _(b):
            cur = lax.rem(b, 3)
            nxt = lax.rem(b + 1, 3)
            pre = lax.rem(b + 2, 3)
            drain_idx(nxt)
            load_idx_async(b + 2, pre)
            for k in range(8):
                wait_gather(k)
                scatter(cur, k)
                issue_gather(nxt, k)

        cur = (nb - 2) % 3
        nxt = (nb - 1) % 3
        drain_idx(nxt)
        for k in range(8):
            wait_gather(k)
            scatter(cur, k)
            issue_gather(nxt, k)

        cur = (nb - 1) % 3
        for k in range(8):
            wait_gather(k)
            scatter(cur, k)

        plsc.subcore_barrier()

        @pl.loop(0, zck_s)
        def _(k):
            r = zbase + k * 128
            pltpu.sync_copy(acc.at[pl.ds(r, 128)],
                            agg_hbm.at[c].at[pl.ds(r, 128)])

    return pass_b


def _dot_t(a, w):
    return lax.dot_general(a, w, (((1,), (1,)), ((), ())),
                           preferred_element_type=jnp.float32)


def _tc_layer_in(h0, wl, wr, bl, n_pad):
    nblk = n_pad // _BLK

    def body(h_ref, wl_ref, wr_ref, bl_ref, p_ref, s_ref):
        h = h_ref[...]
        p = _dot_t(h, wl_ref[...]).astype(jnp.bfloat16)
        p_ref[0] = p[:, :32]
        p_ref[1] = p[:, 32:]
        s_ref[...] = _dot_t(h, wr_ref[...]) + bl_ref[...]

    return pl.pallas_call(
        body,
        grid=(nblk,),
        in_specs=[
            pl.BlockSpec((_BLK, 64), lambda i: (i, 0)),
            pl.BlockSpec((64, 64), lambda i: (0, 0)),
            pl.BlockSpec((64, 64), lambda i: (0, 0)),
            pl.BlockSpec((1, 64), lambda i: (0, 0)),
        ],
        out_specs=[
            pl.BlockSpec((2, _BLK, 32), lambda i: (0, i, 0)),
            pl.BlockSpec((_BLK, 64), lambda i: (i, 0)),
        ],
        out_shape=[
            jax.ShapeDtypeStruct((2, n_pad, 32), jnp.bfloat16),
            jax.ShapeDtypeStruct((n_pad, 64), jnp.float32),
        ],
    )(h0, wl, wr, bl.reshape(1, 64))


def _finish_layer(agg_ref, degp_ref, s_ref):
    a = jnp.concatenate([agg_ref[0], agg_ref[1]], axis=1).astype(jnp.float32)
    deg = degp_ref[0, :, 0:1] + degp_ref[1, :, 0:1]
    inv = 1.0 / jnp.maximum(deg, 1.0)
    return jax.nn.relu(a * inv + s_ref[...])


def _tc_layer_mid(agg3, degp3, s1, wl, wr, bl, n_pad):
    nblk = n_pad // _BLK

    def body(agg_ref, degp_ref, s_ref, wl_ref, wr_ref, bl_ref, p_ref, o_ref):
        h = _finish_layer(agg_ref, degp_ref, s_ref)
        p = _dot_t(h, wl_ref[...]).astype(jnp.bfloat16)
        p_ref[0] = p[:, :32]
        p_ref[1] = p[:, 32:]
        o_ref[...] = _dot_t(h, wr_ref[...]) + bl_ref[...]

    return pl.pallas_call(
        body,
        grid=(nblk,),
        in_specs=[
            pl.BlockSpec((2, _BLK, 32), lambda i: (0, i, 0)),
            pl.BlockSpec((2, _BLK, 16), lambda i: (0, i, 0)),
            pl.BlockSpec((_BLK, 64), lambda i: (i, 0)),
            pl.BlockSpec((64, 64), lambda i: (0, 0)),
            pl.BlockSpec((64, 64), lambda i: (0, 0)),
            pl.BlockSpec((1, 64), lambda i: (0, 0)),
        ],
        out_specs=[
            pl.BlockSpec((2, _BLK, 32), lambda i: (0, i, 0)),
            pl.BlockSpec((_BLK, 64), lambda i: (i, 0)),
        ],
        out_shape=[
            jax.ShapeDtypeStruct((2, n_pad, 32), jnp.bfloat16),
            jax.ShapeDtypeStruct((n_pad, 64), jnp.float32),
        ],
    )(agg3, degp3, s1, wl, wr, bl.reshape(1, 64))


def _tc_pool(agg3, degp3, s2, batch3, wlin_p, blin_p, n_pad, ng, ncp):
    nblk = n_pad // _BLK

    def body(agg_ref, degp_ref, s_ref, b_ref, wlin_ref, blin_ref, o_ref,
             gsum, cnt):
        i = pl.program_id(0)
        h = _finish_layer(agg_ref, degp_ref, s_ref)
        b = b_ref[0]
        oh = (lax.broadcasted_iota(jnp.int32, (ng, _BLK), 0) == b
              ).astype(jnp.float32)

        @pl.when(i == 0)
        def _():
            gsum[...] = jnp.zeros_like(gsum)
            cnt[...] = jnp.zeros_like(cnt)

        gsum[...] += lax.dot_general(oh, h, (((1,), (0,)), ((), ())),
                                     preferred_element_type=jnp.float32)
        cnt[...] += jnp.sum(oh, axis=1, keepdims=True)

        @pl.when(i == nblk - 1)
        def _():
            g = gsum[...] / jnp.maximum(cnt[...], 1.0)
            o_ref[...] = _dot_t(g, wlin_ref[...]) + blin_ref[...]

    return pl.pallas_call(
        body,
        grid=(nblk,),
        in_specs=[
            pl.BlockSpec((2, _BLK, 32), lambda i: (0, i, 0)),
            pl.BlockSpec((2, _BLK, 16), lambda i: (0, i, 0)),
            pl.BlockSpec((_BLK, 64), lambda i: (i, 0)),
            pl.BlockSpec((1, 1, _BLK), lambda i: (i, 0, 0)),
            pl.BlockSpec((ncp, 64), lambda i: (0, 0)),
            pl.BlockSpec((1, ncp), lambda i: (0, 0)),
        ],
        out_specs=pl.BlockSpec((ng, ncp), lambda i: (0, 0)),
        out_shape=jax.ShapeDtypeStruct((ng, ncp), jnp.float32),
        scratch_shapes=[
            pltpu.VMEM((ng, 64), jnp.float32),
            pltpu.VMEM((ng, 1), jnp.float32),
        ],
    )(agg3, degp3, s2, batch3, wlin_p, blin_p)


def _ceil_to(v, m):
    return (v + m - 1) // m * m


def kernel(x, edge_index, batch, embed, Wl1, bl1, Wr1, Wl2, bl2, Wr2,
           Wlin, blin):
    n = x.shape[0]
    e = edge_index.shape[1]
    vocab, emb = embed.shape
    nc = Wlin.shape[0]
    ng = 128

    n_pad = _ceil_to(n, 128 * _NCORE * _NSUB)
    e_pad = _ceil_to(e, 128 * 8 * _NCORE * _NSUB)
    n_chunks = n_pad // 128
    e_chunks = e_pad // 128
    assert n_pad % _BLK == 0

    x_p = jnp.concatenate(
        [x, jnp.arange(n_pad - n, dtype=jnp.int32) % vocab])
    x3d = x_p.reshape(_NCORE * _NSUB, n_chunks // (_NCORE * _NSUB), 128)
    src = edge_index[0]
    dst = edge_index[1]
    src_p = jnp.concatenate(
        [src, jnp.arange(e_pad - e, dtype=jnp.int32) % n])
    dst_p = jnp.concatenate(
        [dst, n + jnp.arange(e_pad - e, dtype=jnp.int32) % (n_pad - n)])
    src2d = src_p.reshape(e_chunks, 128)
    dst2d = dst_p.reshape(e_chunks, 128)
    batch_p = jnp.concatenate([batch, jnp.full((n_pad - n,), ng, jnp.int32)])
    batch3 = batch_p.reshape(n_pad // _BLK, 1, _BLK)

    ones16 = jnp.ones((128, 16), jnp.float32)
    zeros16 = jnp.zeros((128, 16), jnp.float32)
    zeros32 = jnp.zeros((128, 32), jnp.bfloat16)

    ncp = _ceil_to(nc, 16)
    wlin_p = jnp.concatenate([Wlin, jnp.zeros((ncp - nc, 64), jnp.float32)])
    blin_p = jnp.concatenate([blin, jnp.zeros((ncp - nc,), jnp.float32)])
    blin_p = blin_p.reshape(1, ncp)

    pass_a = _make_pass_a(vocab, emb, n_pad, e_chunks)
    h0, degp3 = pass_a(embed, x3d, dst2d, ones16, zeros16)

    pass_b = _make_pass_b(n_pad, e_chunks)

    p1, s1 = _tc_layer_in(h0, Wl1, Wr1, bl1, n_pad)
    agg1_3 = pass_b(p1, src2d, dst2d, zeros32)

    p2, s2 = _tc_layer_mid(agg1_3, degp3, s1, Wl2, Wr2, bl2, n_pad)
    agg2_3 = pass_b(p2, src2d, dst2d, zeros32)

    out_p = _tc_pool(agg2_3, degp3, s2, batch3, wlin_p, blin_p, n_pad, ng, ncp)
    return out_p[:, :nc]


# speedup vs baseline: 11.4738x; 1.0526x over previous
import functools

import jax
import jax.numpy as jnp
from jax import lax
from jax.experimental import pallas as pl
from jax.experimental.pallas import tpu as pltpu
from jax.experimental.pallas import tpu_sc as plsc

_NCORE = 2
_NSUB = 16
_BLK = 4096


def _sc_mesh():
    return plsc.VectorSubcoreMesh(core_axis_name="c", subcore_axis_name="s")


_SC_PARAMS = pltpu.CompilerParams(use_tc_tiling_on_sc=False)


def _make_pass_a(vocab, emb, n_pad, e_chunks):
    n_chunks = n_pad // 128
    nck_w = n_chunks // (_NCORE * _NSUB)
    zck_s = n_pad // _NSUB // 128
    eck_w = e_chunks // (_NCORE * _NSUB)
    assert eck_w % 8 == 0
    eblk = eck_w // 8

    @functools.partial(
        pl.kernel,
        out_type=(
            jax.ShapeDtypeStruct((n_pad, emb), jnp.float32),
            jax.ShapeDtypeStruct((_NCORE, n_pad, 16), jnp.float32),
        ),
        mesh=_sc_mesh(),
        compiler_params=_SC_PARAMS,
        scratch_types=[
            pltpu.VMEM((nck_w, 128), jnp.int32),
            pltpu.VMEM((4, 128, emb), jnp.float32),
            pltpu.VMEM((3, 8, 128), jnp.int32),
            pltpu.VMEM((128, 16), jnp.float32),
            pltpu.VMEM((128, 16), jnp.float32),
            pltpu.VMEM_SHARED((n_pad, 16), jnp.float32),
            pltpu.SemaphoreType.DMA,
            pltpu.SemaphoreType.DMA,
        ],
    )
    def pass_a(embed_hbm, x3d_hbm, dst2d_hbm, ones_hbm, zeros_hbm,
               h0_hbm, degp_hbm, xbuf, rows, dstbuf, onesbuf, zbuf,
               acc, gsem, isem):
        c = lax.axis_index("c")
        s = lax.axis_index("s")
        wid = c * _NSUB + s
        pltpu.sync_copy(ones_hbm, onesbuf)
        pltpu.sync_copy(zeros_hbm, zbuf)

        zbase = s * (n_pad // _NSUB)

        @pl.loop(0, zck_s)
        def _(k):
            pltpu.sync_copy(zbuf, acc.at[pl.ds(zbase + k * 128, 128)])

        nb = wid * nck_w
        pltpu.sync_copy(x3d_hbm.at[wid], xbuf)
        for k in range(min(4, nck_w)):
            pltpu.async_copy(embed_hbm.at[xbuf.at[k]], rows.at[k % 4], gsem)
        for k in range(nck_w):
            pltpu.make_async_copy(embed_hbm.at[pl.ds(0, 128)],
                                  rows.at[k % 4], gsem).wait()
            pltpu.sync_copy(rows.at[k % 4],
                            h0_hbm.at[pl.ds((nb + k) * 128, 128)])
            if k + 4 < nck_w:
                pltpu.async_copy(embed_hbm.at[xbuf.at[k + 4]],
                                 rows.at[k % 4], gsem)

        plsc.subcore_barrier()

        ebase = c * (e_chunks // _NCORE) + s * eck_w

        def load_idx_async(b, slot):
            pltpu.async_copy(dst2d_hbm.at[pl.ds(ebase + b * 8, 8)],
                             dstbuf.at[slot], isem)

        def drain_idx(slot):
            pltpu.make_async_copy(dst2d_hbm.at[pl.ds(0, 8)],
                                  dstbuf.at[slot], isem).wait()

        pltpu.sync_copy(dst2d_hbm.at[pl.ds(ebase, 8)], dstbuf.at[0])
        load_idx_async(1, 1)

        @pl.loop(0, eblk - 2)
        def _(b):
            cur = lax.rem(b, 3)
            drain_idx(lax.rem(b + 1, 3))
            load_idx_async(b + 2, lax.rem(b + 2, 3))
            for k in range(8):
                pltpu.sync_copy(onesbuf, acc.at[dstbuf.at[cur, k]], add=True)

        drain_idx((eblk - 1) % 3)
        for b in (eblk - 2, eblk - 1):
            for k in range(8):
                pltpu.sync_copy(onesbuf, acc.at[dstbuf.at[b % 3, k]],
                                add=True)

        plsc.subcore_barrier()

        @pl.loop(0, zck_s)
        def _(k):
            r = zbase + k * 128
            pltpu.sync_copy(acc.at[pl.ds(r, 128)],
                            degp_hbm.at[c].at[pl.ds(r, 128)])

    return pass_a


def _make_pass_b(n_pad, e_chunks):
    zck_s = n_pad // _NSUB // 128
    eck_s = e_chunks // (_NCORE * _NSUB)
    assert eck_s % 8 == 0
    nb = eck_s // 8

    @functools.partial(
        pl.kernel,
        out_type=jax.ShapeDtypeStruct((_NCORE, n_pad, 64), jnp.bfloat16),
        mesh=_sc_mesh(),
        compiler_params=_SC_PARAMS,
        scratch_types=[
            pltpu.VMEM((3, 8, 128), jnp.int32),
            pltpu.VMEM((3, 8, 128), jnp.int32),
            pltpu.VMEM((4, 128, 64), jnp.bfloat16),
            pltpu.VMEM_SHARED((n_pad, 64), jnp.bfloat16),
            pltpu.SemaphoreType.DMA,
            pltpu.SemaphoreType.DMA,
        ],
    )
    def pass_b(p_hbm, src2d_hbm, dst2d_hbm, zeros_hbm, agg_hbm,
               srcbuf, dstbuf, rows, acc, gsem, isem):
        c = lax.axis_index("c")
        s = lax.axis_index("s")
        zbase = s * (n_pad // _NSUB)

        @pl.loop(0, zck_s)
        def _(k):
            pltpu.sync_copy(zeros_hbm, acc.at[pl.ds(zbase + k * 128, 128)])

        plsc.subcore_barrier()

        base = c * (e_chunks // _NCORE) + s * eck_s

        def load_idx_async(b, slot):
            rowa = base + b * 8
            pltpu.async_copy(src2d_hbm.at[pl.ds(rowa, 8)],
                             srcbuf.at[slot], isem)
            pltpu.async_copy(dst2d_hbm.at[pl.ds(rowa, 8)],
                             dstbuf.at[slot], isem)

        def drain_idx(slot):
            pltpu.make_async_copy(src2d_hbm.at[pl.ds(0, 8)],
                                  srcbuf.at[slot], isem).wait()
            pltpu.make_async_copy(dst2d_hbm.at[pl.ds(0, 8)],
                                  dstbuf.at[slot], isem).wait()

        def issue_gather(slot, k):
            pltpu.async_copy(p_hbm.at[srcbuf.at[slot, k]],
                             rows.at[k % 4], gsem)

        def wait_gather(k):
            pltpu.make_async_copy(p_hbm.at[pl.ds(0, 128)],
                                  rows.at[k % 4], gsem).wait()

        def scatter(slot, k):
            pltpu.sync_copy(rows.at[k % 4], acc.at[dstbuf.at[slot, k]],
                            add=True)

        pltpu.sync_copy(src2d_hbm.at[pl.ds(base, 8)], srcbuf.at[0])
        pltpu.sync_copy(dst2d_hbm.at[pl.ds(base, 8)], dstbuf.at[0])
        load_idx_async(1, 1)
        for k in range(4):
            issue_gather(0, k)

        @pl.loop(0, nb - 2)
        def _(b):
            cur = lax.rem(b, 3)
            nxt = lax.rem(b + 1, 3)
            pre = lax.rem(b + 2, 3)
            drain_idx(nxt)
            load_idx_async(b + 2, pre)
            for k in range(8):
                wait_gather(k)
                scatter(cur, k)
                if k < 4:
                    issue_gather(cur, k + 4)
                else:
                    issue_gather(nxt, k - 4)

        cur = (nb - 2) % 3
        nxt = (nb - 1) % 3
        drain_idx(nxt)
        for k in range(8):
            wait_gather(k)
            scatter(cur, k)
            if k < 4:
                issue_gather(cur, k + 4)
            else:
                issue_gather(nxt, k - 4)

        cur = (nb - 1) % 3
        for k in range(8):
            wait_gather(k)
            scatter(cur, k)
            if k < 4:
                issue_gather(cur, k + 4)

        plsc.subcore_barrier()

        @pl.loop(0, zck_s)
        def _(k):
            r = zbase + k * 128
            pltpu.sync_copy(acc.at[pl.ds(r, 128)],
                            agg_hbm.at[c].at[pl.ds(r, 128)])

    return pass_b


def _dot_t(a, w):
    return lax.dot_general(a, w, (((1,), (1,)), ((), ())),
                           preferred_element_type=jnp.float32)


def _tc_layer_in(h0, wl, wr, bl, n_pad):
    nblk = n_pad // _BLK

    def body(h_ref, wl_ref, wr_ref, bl_ref, p_ref, s_ref):
        h = h_ref[...]
        p_ref[...] = _dot_t(h, wl_ref[...]).astype(jnp.bfloat16)
        s_ref[...] = _dot_t(h, wr_ref[...]) + bl_ref[...]

    return pl.pallas_call(
        body,
        grid=(nblk,),
        in_specs=[
            pl.BlockSpec((_BLK, 64), lambda i: (i, 0)),
            pl.BlockSpec((64, 64), lambda i: (0, 0)),
            pl.BlockSpec((64, 64), lambda i: (0, 0)),
            pl.BlockSpec((1, 64), lambda i: (0, 0)),
        ],
        out_specs=[
            pl.BlockSpec((_BLK, 64), lambda i: (i, 0)),
            pl.BlockSpec((_BLK, 64), lambda i: (i, 0)),
        ],
        out_shape=[
            jax.ShapeDtypeStruct((n_pad, 64), jnp.bfloat16),
            jax.ShapeDtypeStruct((n_pad, 64), jnp.float32),
        ],
    )(h0, wl, wr, bl.reshape(1, 64))


def _finish_layer(agg_ref, degp_ref, s_ref):
    a = (agg_ref[0].astype(jnp.float32) + agg_ref[1].astype(jnp.float32))
    deg = degp_ref[0, :, 0:1] + degp_ref[1, :, 0:1]
    inv = 1.0 / jnp.maximum(deg, 1.0)
    return jax.nn.relu(a * inv + s_ref[...])


def _tc_layer_mid(agg3, degp3, s1, wl, wr, bl, n_pad):
    nblk = n_pad // _BLK

    def body(agg_ref, degp_ref, s_ref, wl_ref, wr_ref, bl_ref, p_ref, o_ref):
        h = _finish_layer(agg_ref, degp_ref, s_ref)
        p_ref[...] = _dot_t(h, wl_ref[...]).astype(jnp.bfloat16)
        o_ref[...] = _dot_t(h, wr_ref[...]) + bl_ref[...]

    return pl.pallas_call(
        body,
        grid=(nblk,),
        in_specs=[
            pl.BlockSpec((2, _BLK, 64), lambda i: (0, i, 0)),
            pl.BlockSpec((2, _BLK, 16), lambda i: (0, i, 0)),
            pl.BlockSpec((_BLK, 64), lambda i: (i, 0)),
            pl.BlockSpec((64, 64), lambda i: (0, 0)),
            pl.BlockSpec((64, 64), lambda i: (0, 0)),
            pl.BlockSpec((1, 64), lambda i: (0, 0)),
        ],
        out_specs=[
            pl.BlockSpec((_BLK, 64), lambda i: (i, 0)),
            pl.BlockSpec((_BLK, 64), lambda i: (i, 0)),
        ],
        out_shape=[
            jax.ShapeDtypeStruct((n_pad, 64), jnp.bfloat16),
            jax.ShapeDtypeStruct((n_pad, 64), jnp.float32),
        ],
    )(agg3, degp3, s1, wl, wr, bl.reshape(1, 64))


def _tc_pool(agg3, degp3, s2, batch3, wlin_p, blin_p, n_pad, ng, ncp):
    nblk = n_pad // _BLK

    def body(agg_ref, degp_ref, s_ref, b_ref, wlin_ref, blin_ref, o_ref,
             gsum, cnt):
        i = pl.program_id(0)
        h = _finish_layer(agg_ref, degp_ref, s_ref)
        b = b_ref[0]
        oh = (lax.broadcasted_iota(jnp.int32, (ng, _BLK), 0) == b
              ).astype(jnp.float32)

        @pl.when(i == 0)
        def _():
            gsum[...] = jnp.zeros_like(gsum)
            cnt[...] = jnp.zeros_like(cnt)

        gsum[...] += lax.dot_general(oh, h, (((1,), (0,)), ((), ())),
                                     preferred_element_type=jnp.float32)
        cnt[...] += jnp.sum(oh, axis=1, keepdims=True)

        @pl.when(i == nblk - 1)
        def _():
            g = gsum[...] / jnp.maximum(cnt[...], 1.0)
            o_ref[...] = _dot_t(g, wlin_ref[...]) + blin_ref[...]

    return pl.pallas_call(
        body,
        grid=(nblk,),
        in_specs=[
            pl.BlockSpec((2, _BLK, 64), lambda i: (0, i, 0)),
            pl.BlockSpec((2, _BLK, 16), lambda i: (0, i, 0)),
            pl.BlockSpec((_BLK, 64), lambda i: (i, 0)),
            pl.BlockSpec((1, 1, _BLK), lambda i: (i, 0, 0)),
            pl.BlockSpec((ncp, 64), lambda i: (0, 0)),
            pl.BlockSpec((1, ncp), lambda i: (0, 0)),
        ],
        out_specs=pl.BlockSpec((ng, ncp), lambda i: (0, 0)),
        out_shape=jax.ShapeDtypeStruct((ng, ncp), jnp.float32),
        scratch_shapes=[
            pltpu.VMEM((ng, 64), jnp.float32),
            pltpu.VMEM((ng, 1), jnp.float32),
        ],
    )(agg3, degp3, s2, batch3, wlin_p, blin_p)


def _ceil_to(v, m):
    return (v + m - 1) // m * m


def kernel(x, edge_index, batch, embed, Wl1, bl1, Wr1, Wl2, bl2, Wr2,
           Wlin, blin):
    n = x.shape[0]
    e = edge_index.shape[1]
    vocab, emb = embed.shape
    nc = Wlin.shape[0]
    ng = 128

    n_pad = _ceil_to(n, 128 * _NCORE * _NSUB)
    e_pad = _ceil_to(e, 128 * 8 * _NCORE * _NSUB)
    n_chunks = n_pad // 128
    e_chunks = e_pad // 128
    assert n_pad % _BLK == 0

    x_p = jnp.concatenate(
        [x, jnp.arange(n_pad - n, dtype=jnp.int32) % vocab])
    x3d = x_p.reshape(_NCORE * _NSUB, n_chunks // (_NCORE * _NSUB), 128)
    src = edge_index[0]
    dst = edge_index[1]
    src_p = jnp.concatenate(
        [src, jnp.arange(e_pad - e, dtype=jnp.int32) % n])
    dst_p = jnp.concatenate(
        [dst, n + jnp.arange(e_pad - e, dtype=jnp.int32) % (n_pad - n)])
    src2d = src_p.reshape(e_chunks, 128)
    dst2d = dst_p.reshape(e_chunks, 128)
    batch_p = jnp.concatenate([batch, jnp.full((n_pad - n,), ng, jnp.int32)])
    batch3 = batch_p.reshape(n_pad // _BLK, 1, _BLK)

    ones16 = jnp.ones((128, 16), jnp.float32)
    zeros16 = jnp.zeros((128, 16), jnp.float32)
    zeros64 = jnp.zeros((128, 64), jnp.bfloat16)

    ncp = _ceil_to(nc, 16)
    wlin_p = jnp.concatenate([Wlin, jnp.zeros((ncp - nc, 64), jnp.float32)])
    blin_p = jnp.concatenate([blin, jnp.zeros((ncp - nc,), jnp.float32)])
    blin_p = blin_p.reshape(1, ncp)

    pass_a = _make_pass_a(vocab, emb, n_pad, e_chunks)
    h0, degp3 = pass_a(embed, x3d, dst2d, ones16, zeros16)

    pass_b = _make_pass_b(n_pad, e_chunks)

    p1, s1 = _tc_layer_in(h0, Wl1, Wr1, bl1, n_pad)
    agg1_3 = pass_b(p1, src2d, dst2d, zeros64)

    p2, s2 = _tc_layer_mid(agg1_3, degp3, s1, Wl2, Wr2, bl2, n_pad)
    agg2_3 = pass_b(p2, src2d, dst2d, zeros64)

    out_p = _tc_pool(agg2_3, degp3, s2, batch3, wlin_p, blin_p, n_pad, ng, ncp)
    return out_p[:, :nc]


# speedup vs baseline: 11.4814x; 1.0007x over previous
import functools

import jax
import jax.numpy as jnp
from jax import lax
from jax.experimental import pallas as pl
from jax.experimental.pallas import tpu as pltpu
from jax.experimental.pallas import tpu_sc as plsc

_NCORE = 2
_NSUB = 16
_BLK = 4096


def _sc_mesh():
    return plsc.VectorSubcoreMesh(core_axis_name="c", subcore_axis_name="s")


_SC_PARAMS = pltpu.CompilerParams(use_tc_tiling_on_sc=False)


def _make_pass_a(vocab, emb, n_pad, e_chunks):
    n_chunks = n_pad // 128
    nck_w = n_chunks // (_NCORE * _NSUB)
    zck_s = n_pad // _NSUB // 128
    eck_w = e_chunks // (_NCORE * _NSUB)
    assert eck_w % 8 == 0
    eblk = eck_w // 8

    @functools.partial(
        pl.kernel,
        out_type=(
            jax.ShapeDtypeStruct((1, n_pad, emb), jnp.float32),
            jax.ShapeDtypeStruct((_NCORE, n_pad, 16), jnp.float32),
        ),
        mesh=_sc_mesh(),
        compiler_params=_SC_PARAMS,
        scratch_types=[
            pltpu.VMEM((nck_w, 128), jnp.int32),
            pltpu.VMEM((4, 128, emb), jnp.float32),
            pltpu.VMEM((3, 8, 128), jnp.int32),
            pltpu.VMEM((128, 16), jnp.float32),
            pltpu.VMEM((128, 16), jnp.float32),
            pltpu.VMEM_SHARED((n_pad, 16), jnp.float32),
            pltpu.SemaphoreType.DMA,
            pltpu.SemaphoreType.DMA,
        ],
    )
    def pass_a(embed_hbm, x3d_hbm, dst2d_hbm, ones_hbm, zeros_hbm,
               h0_hbm, degp_hbm, xbuf, rows, dstbuf, onesbuf, zbuf,
               acc, gsem, isem):
        c = lax.axis_index("c")
        s = lax.axis_index("s")
        wid = c * _NSUB + s
        pltpu.sync_copy(ones_hbm, onesbuf)
        pltpu.sync_copy(zeros_hbm, zbuf)

        zbase = s * (n_pad // _NSUB)

        @pl.loop(0, zck_s)
        def _(k):
            pltpu.sync_copy(zbuf, acc.at[pl.ds(zbase + k * 128, 128)])

        nb = wid * nck_w
        pltpu.sync_copy(x3d_hbm.at[wid], xbuf)
        for k in range(min(4, nck_w)):
            pltpu.async_copy(embed_hbm.at[0].at[xbuf.at[k]],
                             rows.at[k % 4], gsem)
        for k in range(nck_w):
            pltpu.make_async_copy(embed_hbm.at[0].at[pl.ds(0, 128)],
                                  rows.at[k % 4], gsem).wait()
            pltpu.sync_copy(rows.at[k % 4],
                            h0_hbm.at[0].at[pl.ds((nb + k) * 128, 128)])
            if k + 4 < nck_w:
                pltpu.async_copy(embed_hbm.at[0].at[xbuf.at[k + 4]],
                                 rows.at[k % 4], gsem)

        plsc.subcore_barrier()

        ebase = c * (e_chunks // _NCORE) + s * eck_w

        def load_idx_async(b, slot):
            pltpu.async_copy(dst2d_hbm.at[pl.ds(ebase + b * 8, 8)],
                             dstbuf.at[slot], isem)

        def drain_idx(slot):
            pltpu.make_async_copy(dst2d_hbm.at[pl.ds(0, 8)],
                                  dstbuf.at[slot], isem).wait()

        pltpu.sync_copy(dst2d_hbm.at[pl.ds(ebase, 8)], dstbuf.at[0])
        load_idx_async(1, 1)

        @pl.loop(0, eblk - 2)
        def _(b):
            cur = lax.rem(b, 3)
            drain_idx(lax.rem(b + 1, 3))
            load_idx_async(b + 2, lax.rem(b + 2, 3))
            for k in range(8):
                pltpu.sync_copy(onesbuf, acc.at[dstbuf.at[cur, k]], add=True)

        drain_idx((eblk - 1) % 3)
        for b in (eblk - 2, eblk - 1):
            for k in range(8):
                pltpu.sync_copy(onesbuf, acc.at[dstbuf.at[b % 3, k]],
                                add=True)

        plsc.subcore_barrier()

        @pl.loop(0, zck_s)
        def _(k):
            r = zbase + k * 128
            pltpu.sync_copy(acc.at[pl.ds(r, 128)],
                            degp_hbm.at[c].at[pl.ds(r, 128)])

    return pass_a


def _make_pass_b(n_pad, e_chunks):
    zck_s = n_pad // _NSUB // 128
    eck_s = e_chunks // (_NCORE * _NSUB)
    assert eck_s % 8 == 0
    nb = eck_s // 8

    @functools.partial(
        pl.kernel,
        out_type=jax.ShapeDtypeStruct((_NCORE, n_pad, 64), jnp.bfloat16),
        mesh=_sc_mesh(),
        compiler_params=_SC_PARAMS,
        scratch_types=[
            pltpu.VMEM((3, 8, 128), jnp.int32),
            pltpu.VMEM((3, 8, 128), jnp.int32),
            pltpu.VMEM((4, 128, 64), jnp.bfloat16),
            pltpu.VMEM_SHARED((n_pad, 64), jnp.bfloat16),
            pltpu.SemaphoreType.DMA,
            pltpu.SemaphoreType.DMA,
        ],
    )
    def pass_b(p_hbm, src2d_hbm, dst2d_hbm, zeros_hbm, agg_hbm,
               srcbuf, dstbuf, rows, acc, gsem, isem):
        c = lax.axis_index("c")
        s = lax.axis_index("s")
        zbase = s * (n_pad // _NSUB)

        @pl.loop(0, zck_s)
        def _(k):
            pltpu.sync_copy(zeros_hbm, acc.at[pl.ds(zbase + k * 128, 128)])

        plsc.subcore_barrier()

        base = c * (e_chunks // _NCORE) + s * eck_s

        def load_idx_async(b, slot):
            rowa = base + b * 8
            pltpu.async_copy(src2d_hbm.at[pl.ds(rowa, 8)],
                             srcbuf.at[slot], isem)
            pltpu.async_copy(dst2d_hbm.at[pl.ds(rowa, 8)],
                             dstbuf.at[slot], isem)

        def drain_idx(slot):
            pltpu.make_async_copy(src2d_hbm.at[pl.ds(0, 8)],
                                  srcbuf.at[slot], isem).wait()
            pltpu.make_async_copy(dst2d_hbm.at[pl.ds(0, 8)],
                                  dstbuf.at[slot], isem).wait()

        def issue_gather(slot, k):
            pltpu.async_copy(p_hbm.at[0].at[srcbuf.at[slot, k]],
                             rows.at[k % 4], gsem)

        def wait_gather(k):
            pltpu.make_async_copy(p_hbm.at[0].at[pl.ds(0, 128)],
                                  rows.at[k % 4], gsem).wait()

        def scatter(slot, k):
            pltpu.sync_copy(rows.at[k % 4], acc.at[dstbuf.at[slot, k]],
                            add=True)

        pltpu.sync_copy(src2d_hbm.at[pl.ds(base, 8)], srcbuf.at[0])
        pltpu.sync_copy(dst2d_hbm.at[pl.ds(base, 8)], dstbuf.at[0])
        load_idx_async(1, 1)
        for k in range(4):
            issue_gather(0, k)

        @pl.loop(0, nb - 2)
        def _(b):
            cur = lax.rem(b, 3)
            nxt = lax.rem(b + 1, 3)
            pre = lax.rem(b + 2, 3)
            drain_idx(nxt)
            load_idx_async(b + 2, pre)
            for k in range(8):
                wait_gather(k)
                scatter(cur, k)
                if k < 4:
                    issue_gather(cur, k + 4)
                else:
                    issue_gather(nxt, k - 4)

        cur = (nb - 2) % 3
        nxt = (nb - 1) % 3
        drain_idx(nxt)
        for k in range(8):
            wait_gather(k)
            scatter(cur, k)
            if k < 4:
                issue_gather(cur, k + 4)
            else:
                issue_gather(nxt, k - 4)

        cur = (nb - 1) % 3
        for k in range(8):
            wait_gather(k)
            scatter(cur, k)
            if k < 4:
                issue_gather(cur, k + 4)

        plsc.subcore_barrier()

        @pl.loop(0, zck_s)
        def _(k):
            r = zbase + k * 128
            pltpu.sync_copy(acc.at[pl.ds(r, 128)],
                            agg_hbm.at[c].at[pl.ds(r, 128)])

    return pass_b


def _dot_t(a, w):
    return lax.dot_general(a, w, (((1,), (1,)), ((), ())),
                           preferred_element_type=jnp.float32)


def _tc_layer_in(h0, wl, wr, bl, n_pad):
    nblk = n_pad // _BLK

    def body(h_ref, wl_ref, wr_ref, bl_ref, p_ref, s_ref):
        h = h_ref[0]
        p_ref[0] = _dot_t(h, wl_ref[...]).astype(jnp.bfloat16)
        s_ref[...] = _dot_t(h, wr_ref[...]) + bl_ref[...]

    return pl.pallas_call(
        body,
        grid=(nblk,),
        in_specs=[
            pl.BlockSpec((1, _BLK, 64), lambda i: (0, i, 0)),
            pl.BlockSpec((64, 64), lambda i: (0, 0)),
            pl.BlockSpec((64, 64), lambda i: (0, 0)),
            pl.BlockSpec((1, 64), lambda i: (0, 0)),
        ],
        out_specs=[
            pl.BlockSpec((1, _BLK, 64), lambda i: (0, i, 0)),
            pl.BlockSpec((_BLK, 64), lambda i: (i, 0)),
        ],
        out_shape=[
            jax.ShapeDtypeStruct((1, n_pad, 64), jnp.bfloat16),
            jax.ShapeDtypeStruct((n_pad, 64), jnp.float32),
        ],
    )(h0, wl, wr, bl.reshape(1, 64))


def _finish_layer(agg_ref, degp_ref, s_ref):
    a = (agg_ref[0].astype(jnp.float32) + agg_ref[1].astype(jnp.float32))
    deg = degp_ref[0, :, 0:1] + degp_ref[1, :, 0:1]
    inv = 1.0 / jnp.maximum(deg, 1.0)
    return jax.nn.relu(a * inv + s_ref[...])


def _tc_layer_mid(agg3, degp3, s1, wl, wr, bl, n_pad):
    nblk = n_pad // _BLK

    def body(agg_ref, degp_ref, s_ref, wl_ref, wr_ref, bl_ref, p_ref, o_ref):
        h = _finish_layer(agg_ref, degp_ref, s_ref)
        p_ref[0] = _dot_t(h, wl_ref[...]).astype(jnp.bfloat16)
        o_ref[...] = _dot_t(h, wr_ref[...]) + bl_ref[...]

    return pl.pallas_call(
        body,
        grid=(nblk,),
        in_specs=[
            pl.BlockSpec((2, _BLK, 64), lambda i: (0, i, 0)),
            pl.BlockSpec((2, _BLK, 16), lambda i: (0, i, 0)),
            pl.BlockSpec((_BLK, 64), lambda i: (i, 0)),
            pl.BlockSpec((64, 64), lambda i: (0, 0)),
            pl.BlockSpec((64, 64), lambda i: (0, 0)),
            pl.BlockSpec((1, 64), lambda i: (0, 0)),
        ],
        out_specs=[
            pl.BlockSpec((1, _BLK, 64), lambda i: (0, i, 0)),
            pl.BlockSpec((_BLK, 64), lambda i: (i, 0)),
        ],
        out_shape=[
            jax.ShapeDtypeStruct((1, n_pad, 64), jnp.bfloat16),
            jax.ShapeDtypeStruct((n_pad, 64), jnp.float32),
        ],
    )(agg3, degp3, s1, wl, wr, bl.reshape(1, 64))


def _tc_pool(agg3, degp3, s2, batch3, wlin_p, blin_p, n_pad, ng, ncp):
    nblk = n_pad // _BLK

    def body(agg_ref, degp_ref, s_ref, b_ref, wlin_ref, blin_ref, o_ref,
             gsum, cnt):
        i = pl.program_id(0)
        h = _finish_layer(agg_ref, degp_ref, s_ref)
        b = b_ref[0]
        oh = (lax.broadcasted_iota(jnp.int32, (ng, _BLK), 0) == b
              ).astype(jnp.float32)

        @pl.when(i == 0)
        def _():
            gsum[...] = jnp.zeros_like(gsum)
            cnt[...] = jnp.zeros_like(cnt)

        gsum[...] += lax.dot_general(oh, h, (((1,), (0,)), ((), ())),
                                     preferred_element_type=jnp.float32)
        cnt[...] += jnp.sum(oh, axis=1, keepdims=True)

        @pl.when(i == nblk - 1)
        def _():
            g = gsum[...] / jnp.maximum(cnt[...], 1.0)
            o_ref[...] = _dot_t(g, wlin_ref[...]) + blin_ref[...]

    return pl.pallas_call(
        body,
        grid=(nblk,),
        in_specs=[
            pl.BlockSpec((2, _BLK, 64), lambda i: (0, i, 0)),
            pl.BlockSpec((2, _BLK, 16), lambda i: (0, i, 0)),
            pl.BlockSpec((_BLK, 64), lambda i: (i, 0)),
            pl.BlockSpec((1, 1, _BLK), lambda i: (i, 0, 0)),
            pl.BlockSpec((ncp, 64), lambda i: (0, 0)),
            pl.BlockSpec((1, ncp), lambda i: (0, 0)),
        ],
        out_specs=pl.BlockSpec((ng, ncp), lambda i: (0, 0)),
        out_shape=jax.ShapeDtypeStruct((ng, ncp), jnp.float32),
        scratch_shapes=[
            pltpu.VMEM((ng, 64), jnp.float32),
            pltpu.VMEM((ng, 1), jnp.float32),
        ],
    )(agg3, degp3, s2, batch3, wlin_p, blin_p)


def _ceil_to(v, m):
    return (v + m - 1) // m * m


def kernel(x, edge_index, batch, embed, Wl1, bl1, Wr1, Wl2, bl2, Wr2,
           Wlin, blin):
    n = x.shape[0]
    e = edge_index.shape[1]
    vocab, emb = embed.shape
    nc = Wlin.shape[0]
    ng = 128

    n_pad = _ceil_to(n, 128 * _NCORE * _NSUB)
    e_pad = _ceil_to(e, 128 * 8 * _NCORE * _NSUB)
    n_chunks = n_pad // 128
    e_chunks = e_pad // 128
    assert n_pad % _BLK == 0

    x_p = jnp.concatenate(
        [x, jnp.arange(n_pad - n, dtype=jnp.int32) % vocab])
    x3d = x_p.reshape(_NCORE * _NSUB, n_chunks // (_NCORE * _NSUB), 128)
    src = edge_index[0]
    dst = edge_index[1]
    src_p = jnp.concatenate(
        [src, jnp.arange(e_pad - e, dtype=jnp.int32) % n])
    dst_p = jnp.concatenate(
        [dst, n + jnp.arange(e_pad - e, dtype=jnp.int32) % (n_pad - n)])
    src2d = src_p.reshape(e_chunks, 128)
    dst2d = dst_p.reshape(e_chunks, 128)
    batch_p = jnp.concatenate([batch, jnp.full((n_pad - n,), ng, jnp.int32)])
    batch3 = batch_p.reshape(n_pad // _BLK, 1, _BLK)

    ones16 = jnp.ones((128, 16), jnp.float32)
    zeros16 = jnp.zeros((128, 16), jnp.float32)
    zeros64 = jnp.zeros((128, 64), jnp.bfloat16)

    ncp = _ceil_to(nc, 16)
    wlin_p = jnp.concatenate([Wlin, jnp.zeros((ncp - nc, 64), jnp.float32)])
    blin_p = jnp.concatenate([blin, jnp.zeros((ncp - nc,), jnp.float32)])
    blin_p = blin_p.reshape(1, ncp)

    pass_a = _make_pass_a(vocab, emb, n_pad, e_chunks)
    h0, degp3 = pass_a(embed.reshape(1, vocab, emb), x3d, dst2d,
                       ones16, zeros16)

    pass_b = _make_pass_b(n_pad, e_chunks)

    p1, s1 = _tc_layer_in(h0, Wl1, Wr1, bl1, n_pad)
    agg1_3 = pass_b(p1, src2d, dst2d, zeros64)

    p2, s2 = _tc_layer_mid(agg1_3, degp3, s1, Wl2, Wr2, bl2, n_pad)
    agg2_3 = pass_b(p2, src2d, dst2d, zeros64)

    out_p = _tc_pool(agg2_3, degp3, s2, batch3, wlin_p, blin_p, n_pad, ng, ncp)
    return out_p[:, :nc]
